# baseline (speedup 1.0000x reference)
import jax
import jax.numpy as jnp
from jax import lax
from jax.experimental import pallas as pl
from jax.experimental.pallas import tpu as pltpu
from jax.experimental.pallas import tpu_sc as plsc

D = 64
SEQ = 50
NW = 32
SEQ_PER_W = 128
TOK_PER_W = SEQ_PER_W * SEQ
SEQ_PER_CHUNK = 4
TOK_PER_CHUNK = SEQ_PER_CHUNK * SEQ
N_CHUNKS = 32
IDX_ROWS = 56

VCAP = 1000000
NBLK = VCAP // 128
BLK_PER_W = NBLK // NW
BLK_REM = NBLK % NW


def _rsqrt(x):
    i = plsc.bitcast(x, jnp.int32)
    i = jnp.int32(0x5F3759DF) - lax.shift_right_logical(i, 1)
    y = plsc.bitcast(i, jnp.float32)
    for _ in range(2):
        y = y * (1.5 - 0.5 * x * y * y)
    return y


_DNUMS = lax.GatherDimensionNumbers(
    offset_dims=(), collapsed_slice_dims=(0,), start_index_map=(0,))


def _permute(v, perm):
    return lax.gather(v, perm[:, None], _DNUMS, slice_sizes=(1,),
                      mode=lax.GatherScatterMode.PROMISE_IN_BOUNDS)


def _allsum(v):
    for step in (8, 4, 2, 1):
        v = v + _permute(v, jnp.arange(16, dtype=jnp.int32) ^ step)
    return v


def _transpose_body(tokt_hbm, tail_hbm, out_hbm,
                    in_a, in_b, out_a, out_b, tail_v, sem, sem2):
    cid = lax.axis_index("c")
    sid = lax.axis_index("s")
    wid = sid * 2 + cid
    nblk = BLK_PER_W + jnp.where(wid < BLK_REM, 1, 0)
    blk0 = wid * BLK_PER_W + jnp.minimum(wid, BLK_REM)
    lane = lax.iota(jnp.int32, 16)
    rowv = [lane + 16 * (j % 4) for j in range(8)]

    def fetch(b, in_v, s):
        return pltpu.async_copy(
            tokt_hbm.at[pl.ds(0, D), pl.ds(b * 128, 128)], in_v, s)

    def fetch_wait(b, in_v, s):
        pltpu.make_async_copy(
            tokt_hbm.at[pl.ds(0, D), pl.ds(b * 128, 128)], in_v, s).wait()

    def put(b, out_v):
        pltpu.sync_copy(out_v, out_hbm.at[pl.ds(b * 64, 64)])

    def transpose(in_v, out_v):
        def pr_body(pr, _):
            base = 2 * pr
            for j in range(8):
                colv = jnp.full((16,), base + j // 4, jnp.int32)
                out_v[pr, pl.ds(j * 16, 16)] = plsc.load_gather(
                    in_v, [rowv[j], colv])
            return 0

        lax.fori_loop(0, 64, pr_body, 0)

    fetch(blk0, in_a, sem)

    def pair_body(i, _):
        b = blk0 + i * 2

        @pl.when(i * 2 + 1 < nblk)
        def _():
            fetch(b + 1, in_b, sem2)

        fetch_wait(b, in_a, sem)
        transpose(in_a, out_a)
        put(b, out_a)

        @pl.when(i * 2 + 2 < nblk)
        def _():
            fetch(b + 2, in_a, sem)

        @pl.when(i * 2 + 1 < nblk)
        def _():
            fetch_wait(b + 1, in_b, sem2)
            transpose(in_b, out_b)
            put(b + 1, out_b)

        return 0

    lax.fori_loop(0, (nblk + 1) // 2, pair_body, 0)

    @pl.when(wid == NW - 1)
    def _():
        pltpu.sync_copy(tail_hbm, tail_v)
        pltpu.sync_copy(tail_v, out_hbm.at[pl.ds(NBLK * 64, 32)])


def _embed_body(x_hbm, tok_hbm, pos_hbm, wb_hbm, out_hbm,
                idx_v, pair_v, par_v, rows_a, rows_b, packed_v, pos_v, wb_v,
                sem_a, sem_b):
    cid = lax.axis_index("c")
    sid = lax.axis_index("s")
    wid = sid * 2 + cid

    pltpu.sync_copy(pos_hbm, pos_v)
    pltpu.sync_copy(wb_hbm, wb_v)
    roff = lax.rem(wid * (TOK_PER_W // 128), 8)
    row0 = pl.multiple_of(wid * (TOK_PER_W // 128) - roff, 8)
    pltpu.sync_copy(x_hbm.at[pl.ds(row0, IDX_ROWS)], idx_v)

    def split_body(r, _):
        for j in range(8):
            ids = idx_v[r, pl.ds(j * 16, 16)]
            par_v[pl.ds(r * 128 + j * 16, 16)] = lax.bitwise_and(ids, 1)
            pair_v[pl.ds(r * 128 + j * 16, 16)] = (
                lax.shift_right_logical(ids, 1))
        return 0

    lax.fori_loop(0, IDX_ROWS, split_body, 0)
    ebase = roff * 128

    lw = [wb_v[pl.ds(k * 16, 16)] for k in range(4)]
    lb = [wb_v[pl.ds(64 + k * 16, 16)] for k in range(4)]
    lane = lax.iota(jnp.int32, 16)

    def idx_slice(c):
        return pair_v.at[pl.ds(ebase + c * TOK_PER_CHUNK, TOK_PER_CHUNK)]

    def start_gather(c, rows_v, sem):
        return pltpu.async_copy(tok_hbm.at[idx_slice(c)], rows_v, sem)

    def gather_wait(c, rows_v, sem):
        pltpu.make_async_copy(tok_hbm.at[idx_slice(c)], rows_v, sem).wait()

    def compute_chunk(c, half, rows_v):
        def s_body(s, _):
            sh = lax.shift_right_logical(s, 1)
            colb = lax.bitwise_and(s, 1) * D
            psel = colb > 0
            p = [jnp.where(psel,
                           pos_v[sh, pl.ds(D + k * 16, 16)],
                           pos_v[sh, pl.ds(k * 16, 16)]) for k in range(4)]
            row16 = ebase + c * TOK_PER_CHUNK + jnp.minimum(
                lane * SEQ + s, TOK_PER_CHUNK - 1)
            pars = plsc.load_gather(par_v, [row16])
            for q in range(SEQ_PER_CHUNK):
                t = q * SEQ + s
                sel = _permute(pars, jnp.full((16,), q, jnp.int32)) > 0
                e = []
                for k in range(4):
                    lo = rows_v[t, pl.ds(k * 16, 16)]
                    hi = rows_v[t, pl.ds(D + k * 16, 16)]
                    e.append(jnp.where(sel, hi, lo) + p[k])
                tot = _allsum((e[0] + e[1]) + (e[2] + e[3]))
                tot2 = _allsum((e[0] * e[0] + e[1] * e[1])
                               + (e[2] * e[2] + e[3] * e[3]))
                mean = tot * (1.0 / D)
                var = tot2 * (1.0 / D) - mean * mean
                rstd = _rsqrt(var + 1e-5)
                prow = half * (TOK_PER_CHUNK // 2) + q * (SEQ // 2) + sh
                for k in range(4):
                    packed_v[prow, pl.ds(colb + k * 16, 16)] = (
                        (e[k] - mean) * rstd * lw[k] + lb[k])
            return 0

        lax.fori_loop(0, SEQ, s_body, 0)

    start_gather(0, rows_a, sem_a)

    def pair_body(i, _):
        c0 = i * 2
        start_gather(c0 + 1, rows_b, sem_b)
        gather_wait(c0, rows_a, sem_a)
        compute_chunk(c0, 0, rows_a)

        @pl.when(i < N_CHUNKS // 2 - 1)
        def _():
            start_gather(c0 + 2, rows_a, sem_a)

        gather_wait(c0 + 1, rows_b, sem_b)
        compute_chunk(c0 + 1, 1, rows_b)
        pbase = pl.multiple_of((wid * (N_CHUNKS // 2) + i) * TOK_PER_CHUNK, 8)
        pltpu.sync_copy(packed_v, out_hbm.at[pl.ds(pbase, TOK_PER_CHUNK)])
        return 0

    lax.fori_loop(0, N_CHUNKS // 2, pair_body, 0)


_MESH = dict(core_axis_name="c", subcore_axis_name="s")


def kernel(x, tok_table, pos_table, ln_w, ln_b):
    batch, seq = x.shape
    n_tok = batch * seq
    vocab = tok_table.shape[0]

    transpose_run = pl.kernel(
        _transpose_body,
        out_type=jax.ShapeDtypeStruct((vocab // 2, 2 * D), jnp.float32),
        mesh=plsc.VectorSubcoreMesh(**_MESH),
        compiler_params=pltpu.CompilerParams(needs_layout_passes=False),
        scratch_types=[
            pltpu.VMEM((D, 128), jnp.float32),
            pltpu.VMEM((D, 128), jnp.float32),
            pltpu.VMEM((D, 128), jnp.float32),
            pltpu.VMEM((D, 128), jnp.float32),
            pltpu.VMEM((32, 128), jnp.float32),
            pltpu.SemaphoreType.DMA,
            pltpu.SemaphoreType.DMA,
        ],
    )
    embed_run = pl.kernel(
        _embed_body,
        out_type=jax.ShapeDtypeStruct((n_tok // 2, 2 * D), jnp.float32),
        mesh=plsc.VectorSubcoreMesh(**_MESH),
        compiler_params=pltpu.CompilerParams(needs_layout_passes=False),
        scratch_types=[
            pltpu.VMEM((IDX_ROWS, 128), jnp.int32),
            pltpu.VMEM((IDX_ROWS * 128,), jnp.int32),
            pltpu.VMEM((IDX_ROWS * 128,), jnp.int32),
            pltpu.VMEM((TOK_PER_CHUNK, 2 * D), jnp.float32),
            pltpu.VMEM((TOK_PER_CHUNK, 2 * D), jnp.float32),
            pltpu.VMEM((TOK_PER_CHUNK, 2 * D), jnp.float32),
            pltpu.VMEM((32, 2 * D), jnp.float32),
            pltpu.VMEM((2 * D,), jnp.float32),
            pltpu.SemaphoreType.DMA,
            pltpu.SemaphoreType.DMA,
        ],
    )

    tail = tok_table[NBLK * 128:].reshape(32, 2 * D)
    pairs = transpose_run(tok_table.T, tail)
    pos2 = pos_table.reshape(pos_table.shape[0] // 2, 2 * D)[:32]
    wb = jnp.concatenate([ln_w, ln_b])
    out = embed_run(x.reshape(n_tok // 128, 128), pairs, pos2, wb)
    return out.reshape(batch, seq, D)


# speedup vs baseline: 1.0440x; 1.0440x over previous
import jax
import jax.numpy as jnp
from jax import lax
from jax.experimental import pallas as pl
from jax.experimental.pallas import tpu as pltpu
from jax.experimental.pallas import tpu_sc as plsc

D = 64
SEQ = 50
NW = 32
SEQ_PER_W = 128
TOK_PER_W = SEQ_PER_W * SEQ
SEQ_PER_CHUNK = 4
TOK_PER_CHUNK = SEQ_PER_CHUNK * SEQ
N_CHUNKS = 32
IDX_ROWS = 56

VCAP = 1000000
NBLK = VCAP // 128
BLK_PER_W = NBLK // NW
BLK_REM = NBLK % NW


def _rsqrt(x):
    i = plsc.bitcast(x, jnp.int32)
    i = jnp.int32(0x5F3759DF) - lax.shift_right_logical(i, 1)
    y = plsc.bitcast(i, jnp.float32)
    for _ in range(2):
        y = y * (1.5 - 0.5 * x * y * y)
    return y


_DNUMS = lax.GatherDimensionNumbers(
    offset_dims=(), collapsed_slice_dims=(0,), start_index_map=(0,))


def _permute(v, perm):
    return lax.gather(v, perm[:, None], _DNUMS, slice_sizes=(1,),
                      mode=lax.GatherScatterMode.PROMISE_IN_BOUNDS)


def _allsum(v):
    for step in (8, 4, 2, 1):
        v = v + _permute(v, jnp.arange(16, dtype=jnp.int32) ^ step)
    return v


def _transpose_body(tokt_hbm, tail_hbm, out_hbm,
                    in_a, in_b, out_a, out_b, tail_v, sem, sem2, psem, psem2):
    cid = lax.axis_index("c")
    sid = lax.axis_index("s")
    wid = sid * 2 + cid
    nblk = BLK_PER_W + jnp.where(wid < BLK_REM, 1, 0)
    blk0 = wid * BLK_PER_W + jnp.minimum(wid, BLK_REM)
    lane = lax.iota(jnp.int32, 16)
    rowv = [lane + 16 * (j % 4) for j in range(8)]

    def fetch(b, in_v, s):
        return pltpu.async_copy(
            tokt_hbm.at[pl.ds(0, D), pl.ds(b * 128, 128)], in_v, s)

    def fetch_wait(b, in_v, s):
        pltpu.make_async_copy(
            tokt_hbm.at[pl.ds(0, D), pl.ds(b * 128, 128)], in_v, s).wait()

    def put_start(b, out_v, s):
        pltpu.async_copy(out_v, out_hbm.at[pl.ds(b * 64, 64)], s)

    def put_drain(s):
        pltpu.make_async_copy(
            tokt_hbm.at[pl.ds(0, D), pl.ds(0, 128)], in_a, s).wait()

    def transpose(in_v, out_v):
        def pr_body(g, _):
            for u in range(4):
                pr = g * 4 + u
                base = 2 * pr
                colv0 = jnp.full((16,), base, jnp.int32)
                colv1 = jnp.full((16,), base + 1, jnp.int32)
                for j in range(8):
                    out_v[pr, pl.ds(j * 16, 16)] = plsc.load_gather(
                        in_v, [rowv[j], colv1 if j >= 4 else colv0])
            return 0

        lax.fori_loop(0, 16, pr_body, 0)

    fetch(blk0, in_a, sem)

    def pair_body(i, _):
        b = blk0 + i * 2

        @pl.when(i * 2 + 1 < nblk)
        def _():
            fetch(b + 1, in_b, sem2)

        fetch_wait(b, in_a, sem)

        @pl.when(i > 0)
        def _():
            put_drain(psem)

        transpose(in_a, out_a)
        put_start(b, out_a, psem)

        @pl.when(i * 2 + 2 < nblk)
        def _():
            fetch(b + 2, in_a, sem)

        @pl.when(i * 2 + 1 < nblk)
        def _():
            fetch_wait(b + 1, in_b, sem2)

            @pl.when(i > 0)
            def _():
                put_drain(psem2)

            transpose(in_b, out_b)
            put_start(b + 1, out_b, psem2)

        return 0

    lax.fori_loop(0, (nblk + 1) // 2, pair_body, 0)
    put_drain(psem)

    @pl.when(nblk > 1)
    def _():
        put_drain(psem2)

    @pl.when(wid == NW - 1)
    def _():
        pltpu.sync_copy(tail_hbm, tail_v)
        pltpu.sync_copy(tail_v, out_hbm.at[pl.ds(NBLK * 64, 32)])


def _embed_body(x_hbm, tok_hbm, pos_hbm, wb_hbm, out_hbm,
                idx_v, pair_v, par_v, rows_a, rows_b, packed_v, pos_v, wb_v,
                sem_a, sem_b):
    cid = lax.axis_index("c")
    sid = lax.axis_index("s")
    wid = sid * 2 + cid

    pltpu.sync_copy(pos_hbm, pos_v)
    pltpu.sync_copy(wb_hbm, wb_v)
    roff = lax.rem(wid * (TOK_PER_W // 128), 8)
    row0 = pl.multiple_of(wid * (TOK_PER_W // 128) - roff, 8)
    pltpu.sync_copy(x_hbm.at[pl.ds(row0, IDX_ROWS)], idx_v)

    def split_body(r, _):
        for j in range(8):
            ids = idx_v[r, pl.ds(j * 16, 16)]
            par_v[pl.ds(r * 128 + j * 16, 16)] = lax.bitwise_and(ids, 1)
            pair_v[pl.ds(r * 128 + j * 16, 16)] = (
                lax.shift_right_logical(ids, 1))
        return 0

    lax.fori_loop(0, IDX_ROWS, split_body, 0)
    ebase = roff * 128

    lw = [wb_v[pl.ds(k * 16, 16)] for k in range(4)]
    lb = [wb_v[pl.ds(64 + k * 16, 16)] for k in range(4)]
    lane = lax.iota(jnp.int32, 16)

    def idx_slice(c):
        return pair_v.at[pl.ds(ebase + c * TOK_PER_CHUNK, TOK_PER_CHUNK)]

    def start_gather(c, rows_v, sem):
        return pltpu.async_copy(tok_hbm.at[idx_slice(c)], rows_v, sem)

    def gather_wait(c, rows_v, sem):
        pltpu.make_async_copy(tok_hbm.at[idx_slice(c)], rows_v, sem).wait()

    def compute_chunk(c, half, rows_v):
        def s_body(s, _):
            sh = lax.shift_right_logical(s, 1)
            colb = lax.bitwise_and(s, 1) * D
            psel = colb > 0
            p = [jnp.where(psel,
                           pos_v[sh, pl.ds(D + k * 16, 16)],
                           pos_v[sh, pl.ds(k * 16, 16)]) for k in range(4)]
            row16 = ebase + c * TOK_PER_CHUNK + jnp.minimum(
                lane * SEQ + s, TOK_PER_CHUNK - 1)
            pars = plsc.load_gather(par_v, [row16])
            for q in range(SEQ_PER_CHUNK):
                t = q * SEQ + s
                sel = _permute(pars, jnp.full((16,), q, jnp.int32)) > 0
                e = []
                for k in range(4):
                    lo = rows_v[t, pl.ds(k * 16, 16)]
                    hi = rows_v[t, pl.ds(D + k * 16, 16)]
                    e.append(jnp.where(sel, hi, lo) + p[k])
                tot = _allsum((e[0] + e[1]) + (e[2] + e[3]))
                tot2 = _allsum((e[0] * e[0] + e[1] * e[1])
                               + (e[2] * e[2] + e[3] * e[3]))
                mean = tot * (1.0 / D)
                var = tot2 * (1.0 / D) - mean * mean
                rstd = _rsqrt(var + 1e-5)
                prow = half * (TOK_PER_CHUNK // 2) + q * (SEQ // 2) + sh
                for k in range(4):
                    packed_v[prow, pl.ds(colb + k * 16, 16)] = (
                        (e[k] - mean) * rstd * lw[k] + lb[k])
            return 0

        lax.fori_loop(0, SEQ, s_body, 0)

    start_gather(0, rows_a, sem_a)

    def pair_body(i, _):
        c0 = i * 2
        start_gather(c0 + 1, rows_b, sem_b)
        gather_wait(c0, rows_a, sem_a)
        compute_chunk(c0, 0, rows_a)

        @pl.when(i < N_CHUNKS // 2 - 1)
        def _():
            start_gather(c0 + 2, rows_a, sem_a)

        gather_wait(c0 + 1, rows_b, sem_b)
        compute_chunk(c0 + 1, 1, rows_b)
        pbase = pl.multiple_of((wid * (N_CHUNKS // 2) + i) * TOK_PER_CHUNK, 8)
        pltpu.sync_copy(packed_v, out_hbm.at[pl.ds(pbase, TOK_PER_CHUNK)])
        return 0

    lax.fori_loop(0, N_CHUNKS // 2, pair_body, 0)


_MESH = dict(core_axis_name="c", subcore_axis_name="s")


def kernel(x, tok_table, pos_table, ln_w, ln_b):
    batch, seq = x.shape
    n_tok = batch * seq
    vocab = tok_table.shape[0]

    transpose_run = pl.kernel(
        _transpose_body,
        out_type=jax.ShapeDtypeStruct((vocab // 2, 2 * D), jnp.float32),
        mesh=plsc.VectorSubcoreMesh(**_MESH),
        compiler_params=pltpu.CompilerParams(needs_layout_passes=False),
        scratch_types=[
            pltpu.VMEM((D, 128), jnp.float32),
            pltpu.VMEM((D, 128), jnp.float32),
            pltpu.VMEM((D, 128), jnp.float32),
            pltpu.VMEM((D, 128), jnp.float32),
            pltpu.VMEM((32, 128), jnp.float32),
            pltpu.SemaphoreType.DMA,
            pltpu.SemaphoreType.DMA,
            pltpu.SemaphoreType.DMA,
            pltpu.SemaphoreType.DMA,
        ],
    )
    embed_run = pl.kernel(
        _embed_body,
        out_type=jax.ShapeDtypeStruct((n_tok // 2, 2 * D), jnp.float32),
        mesh=plsc.VectorSubcoreMesh(**_MESH),
        compiler_params=pltpu.CompilerParams(needs_layout_passes=False),
        scratch_types=[
            pltpu.VMEM((IDX_ROWS, 128), jnp.int32),
            pltpu.VMEM((IDX_ROWS * 128,), jnp.int32),
            pltpu.VMEM((IDX_ROWS * 128,), jnp.int32),
            pltpu.VMEM((TOK_PER_CHUNK, 2 * D), jnp.float32),
            pltpu.VMEM((TOK_PER_CHUNK, 2 * D), jnp.float32),
            pltpu.VMEM((TOK_PER_CHUNK, 2 * D), jnp.float32),
            pltpu.VMEM((32, 2 * D), jnp.float32),
            pltpu.VMEM((2 * D,), jnp.float32),
            pltpu.SemaphoreType.DMA,
            pltpu.SemaphoreType.DMA,
        ],
    )

    tail = tok_table[NBLK * 128:].reshape(32, 2 * D)
    pairs = transpose_run(tok_table.T, tail)
    pos2 = pos_table.reshape(pos_table.shape[0] // 2, 2 * D)[:32]
    wb = jnp.concatenate([ln_w, ln_b])
    out = embed_run(x.reshape(n_tok // 128, 128), pairs, pos2, wb)
    return out.reshape(batch, seq, D)


# speedup vs baseline: 2.3170x; 2.2193x over previous
import jax
import jax.numpy as jnp
from jax import lax
from jax.experimental import pallas as pl
from jax.experimental.pallas import tpu as pltpu
from jax.experimental.pallas import tpu_sc as plsc

D = 64
SEQ = 50
NW = 32
SEQ_PER_CHUNK = 16
TOK_PER_CHUNK = SEQ_PER_CHUNK * SEQ


def _rsqrt(x):
    i = plsc.bitcast(x, jnp.int32)
    i = jnp.int32(0x5F3759DF) - lax.shift_right_logical(i, 1)
    y = plsc.bitcast(i, jnp.float32)
    for _ in range(2):
        y = y * (1.5 - 0.5 * x * y * y)
    return y


_DNUMS = lax.GatherDimensionNumbers(
    offset_dims=(), collapsed_slice_dims=(0,), start_index_map=(0,))


def _allsum(v):
    for step in (8, 4, 2, 1):
        perm = jnp.arange(16, dtype=jnp.int32) ^ step
        v = v + lax.gather(v, perm[:, None], _DNUMS, slice_sizes=(1,),
                           mode=lax.GatherScatterMode.PROMISE_IN_BOUNDS)
    return v


def _body(x_hbm, tok_hbm, pos_hbm, lnw_hbm, lnb_hbm, out_hbm,
          idx_v, rows_v, pos_v, lnw_v, lnb_v, sem):
    cid = lax.axis_index("c")
    sid = lax.axis_index("s")
    wid = sid * 2 + cid
    n_chunks = x_hbm.shape[0] // (NW * TOK_PER_CHUNK)

    pltpu.sync_copy(pos_hbm.at[pl.ds(0, 56)], pos_v)
    pltpu.sync_copy(lnw_hbm, lnw_v)
    pltpu.sync_copy(lnb_hbm, lnb_v)

    lw = [lnw_v[pl.ds(k * 16, 16)] for k in range(4)]
    lb = [lnb_v[pl.ds(k * 16, 16)] for k in range(4)]

    def chunk_body(c, _):
        tok_base = pl.multiple_of((wid * n_chunks + c) * TOK_PER_CHUNK, 8)
        pltpu.sync_copy(x_hbm.at[pl.ds(tok_base, TOK_PER_CHUNK)], idx_v)
        pltpu.async_copy(tok_hbm.at[idx_v], rows_v, sem).wait()

        def s_body(s, _):
            p = [pos_v[s, pl.ds(k * 16, 16)] for k in range(4)]

            def q_body(q, _):
                t = q * SEQ + s
                e = [rows_v[t, pl.ds(k * 16, 16)] + p[k] for k in range(4)]
                tot = _allsum((e[0] + e[1]) + (e[2] + e[3]))
                tot2 = _allsum((e[0] * e[0] + e[1] * e[1])
                               + (e[2] * e[2] + e[3] * e[3]))
                mean = tot * (1.0 / D)
                var = tot2 * (1.0 / D) - mean * mean
                rstd = _rsqrt(var + 1e-5)
                for k in range(4):
                    rows_v[t, pl.ds(k * 16, 16)] = (
                        (e[k] - mean) * rstd * lw[k] + lb[k])
                return 0

            lax.fori_loop(0, SEQ_PER_CHUNK, q_body, 0, unroll=4)
            return 0

        lax.fori_loop(0, SEQ, s_body, 0)
        pltpu.sync_copy(rows_v, out_hbm.at[pl.ds(tok_base, TOK_PER_CHUNK)])
        return 0

    lax.fori_loop(0, n_chunks, chunk_body, 0)


def kernel(x, tok_table, pos_table, ln_w, ln_b):
    batch, seq = x.shape
    n_tok = batch * seq
    run = pl.kernel(
        _body,
        out_type=jax.ShapeDtypeStruct((n_tok, D), jnp.float32),
        mesh=plsc.VectorSubcoreMesh(core_axis_name="c", subcore_axis_name="s"),
        compiler_params=pltpu.CompilerParams(
            needs_layout_passes=False, use_tc_tiling_on_sc=False),
        scratch_types=[
            pltpu.VMEM((TOK_PER_CHUNK,), jnp.int32),
            pltpu.VMEM((TOK_PER_CHUNK, D), jnp.float32),
            pltpu.VMEM((56, D), jnp.float32),
            pltpu.VMEM((D,), jnp.float32),
            pltpu.VMEM((D,), jnp.float32),
            pltpu.SemaphoreType.DMA,
        ],
    )
    out = run(x.reshape(n_tok), tok_table, pos_table, ln_w, ln_b)
    return out.reshape(batch, seq, D)
